# Initial kernel scaffold; baseline (speedup 1.0000x reference)
import jax
import jax.numpy as jnp
from jax.experimental import pallas as pl


def kernel(x, table, W, b):
    raise NotImplementedError("write your pallas kernel here")


# speedup vs baseline: 66.5763x; 66.5763x over previous
import functools

import jax
import jax.numpy as jnp
from jax import lax
from jax.experimental import pallas as pl
from jax.experimental.pallas import tpu as pltpu
from jax.experimental.pallas import tpu_sc as plsc

_VOCAB = 1000
_BATCH = 16384
_SEQ = 50
_DIM = 64
_LUT = 1024

_NC = 2
_NS = 16
_NW = _NC * _NS
_BPW = _BATCH // _NW
_L = 16


def _lut_body(table_ref, w_ref, b_ref, out_ref):
    wd = w_ref[1:2, :] - w_ref[0:1, :]
    d = jax.lax.dot_general(
        table_ref[:, :], wd, (((1,), (1,)), ((), ())),
        preferred_element_type=jnp.float32)
    db = b_ref[0:1, 1:2] - b_ref[0:1, 0:1]
    dfull = jnp.concatenate(
        [d, jnp.zeros((_LUT - _VOCAB, 1), jnp.float32)], axis=0)
    out_ref[:, :] = (dfull + db) * (1.0 / _SEQ)


_lut_call = pl.pallas_call(
    _lut_body,
    out_shape=jax.ShapeDtypeStruct((_LUT, 1), jnp.float32),
)


def _sc_body(x_hbm, d_hbm, out_hbm, x_v, d_v, out_v):
    wid = lax.axis_index("s") * _NC + lax.axis_index("c")
    pltpu.sync_copy(x_hbm.at[pl.ds(wid * (_BPW * _SEQ), _BPW * _SEQ)], x_v)
    pltpu.sync_copy(d_hbm, d_v)

    iota = lax.iota(jnp.int32, _L)
    iota_s = iota * _SEQ
    iota_2 = iota * 2

    def body(g, carry):
        idx0 = iota_s + g * (_L * _SEQ)
        acc = jnp.zeros((_L,), jnp.float32)
        for s in range(_SEQ):
            xi = plsc.load_gather(x_v, [idx0 + s])
            acc = acc + plsc.load_gather(d_v, [xi])
        p1 = 1.0 / (1.0 + jnp.exp(-acc))
        o = iota_2 + g * (2 * _L)
        plsc.store_scatter(out_v, [o], 1.0 - p1)
        plsc.store_scatter(out_v, [o + 1], p1)
        return carry

    lax.fori_loop(0, _BPW // _L, body, 0)
    pltpu.sync_copy(out_v, out_hbm.at[pl.ds(wid * (2 * _BPW), 2 * _BPW)])


_sc_call = functools.partial(
    pl.kernel,
    out_type=jax.ShapeDtypeStruct((2 * _BATCH,), jnp.float32),
    mesh=plsc.VectorSubcoreMesh(core_axis_name="c", subcore_axis_name="s"),
    scratch_types=[
        pltpu.VMEM((_BPW * _SEQ,), jnp.int32),
        pltpu.VMEM((_LUT,), jnp.float32),
        pltpu.VMEM((2 * _BPW,), jnp.float32),
    ],
    compiler_params=pltpu.CompilerParams(needs_layout_passes=False),
)(_sc_body)


def kernel(x, table, W, b):
    d = _lut_call(table, W, b.reshape(1, 2))
    out = _sc_call(x.reshape(-1), d.reshape(_LUT))
    return out.reshape(_BATCH, 2)
